# baseline (speedup 1.0000x reference)
import functools

import jax
import jax.numpy as jnp
from jax import lax
from jax.experimental import pallas as pl
from jax.experimental.pallas import tpu as pltpu
from jax.experimental.pallas import tpu_sc as plsc

_D = 2048
_KEEP = max(1, int(_D * 0.5))
_ROWS_PER_BLOCK = 1024

_NC = 1
_NS = 16
_L = 16
_NW = _NC * _NS
_DPW = _D // _NW
_NDV = _DPW // _L


_NCHUNK = _D // _L


@functools.partial(
    pl.kernel,
    out_type=jax.ShapeDtypeStruct((_D,), jnp.float32),
    mesh=plsc.VectorSubcoreMesh(core_axis_name="c", subcore_axis_name="s",
                                num_cores=_NC),
    scratch_types=[
        pltpu.VMEM((_D,), jnp.float32),
        pltpu.VMEM((_D,), jnp.uint32),
        pltpu.VMEM((_DPW,), jnp.float32),
    ],
)
def _sc_mask(imp_hbm, out_hbm, imp_v, key_v, out_v):
    wid = lax.axis_index("s") * _NC + lax.axis_index("c")
    base = wid * _DPW
    pltpu.sync_copy(imp_hbm, imp_v)

    lane = lax.iota(jnp.int32, _L)

    def vsum(v):
        parts = [v[l] for l in range(_L)]
        while len(parts) > 1:
            parts = [parts[i] + parts[i + 1]
                     for i in range(0, len(parts) - 1, 2)] + (
                         [parts[-1]] if len(parts) % 2 else [])
        return parts[0]

    def to_key(c, _):
        v = imp_v[pl.ds(c * _L, _L)] + 0.0
        b = lax.bitcast_convert_type(v, jnp.int32)
        ks = b ^ (jnp.uint32(0x7FFFFFFF).astype(jnp.int32) & (b >> 31))
        key_v[pl.ds(c * _L, _L)] = lax.bitcast_convert_type(
            ks, jnp.uint32) ^ jnp.uint32(0x80000000)
        return 0

    lax.fori_loop(0, _NCHUNK, to_key, 0)

    _UNROLL = 8

    def count_chunks(hit_fn):
        def cbody(i, accs):
            return tuple(
                accs[u] + jnp.where(hit_fn(i * _UNROLL + u), 1, 0).astype(
                    jnp.int32) for u in range(_UNROLL))

        accs = lax.fori_loop(0, _NCHUNK // _UNROLL, cbody,
                             (jnp.zeros((_L,), jnp.int32),) * _UNROLL)
        accs = list(accs)
        while len(accs) > 1:
            accs = [accs[i] + accs[i + 1] for i in range(0, len(accs), 2)]
        return vsum(accs[0])

    def count_ge(thr):
        thr_b = jnp.full((_L,), thr, jnp.uint32)
        return count_chunks(lambda c: key_v[pl.ds(c * _L, _L)] >= thr_b)

    def round_(r, prefix):
        bit = 31 - r
        cand = prefix | (jnp.uint32(1) << bit.astype(jnp.uint32))
        cnt = count_ge(cand)
        return jnp.where(cnt >= _KEEP, cand, prefix)

    thr = lax.fori_loop(0, 32, round_, jnp.uint32(0))

    thr_b = jnp.full((_L,), thr, jnp.uint32)

    cnt_gt = count_chunks(lambda c: key_v[pl.ds(c * _L, _L)] > thr_b)
    quota = _KEEP - cnt_gt

    def count_eq_below(c):
        c_b = jnp.full((_L,), c, jnp.int32)

        def hit(ch):
            kc = key_v[pl.ds(ch * _L, _L)]
            eidx = lane + ch * _L
            return (kc == thr_b) & (eidx < c_b)

        return count_chunks(hit)

    def idx_round(r, cmax):
        bit = 10 - r
        cand = cmax | (jnp.int32(1) << bit)
        cnt = count_eq_below(cand)
        return jnp.where(cnt < quota, cand, cmax)

    cmax = lax.fori_loop(0, 11, idx_round, jnp.int32(0))

    cmax_b = jnp.full((_L,), cmax, jnp.int32)
    for dv in range(_NDV):
        kdv = key_v[pl.ds(base + dv * _L, _L)]
        didx = lane + (base + dv * _L)
        sel = (kdv > thr_b) | ((kdv == thr_b) & (didx <= cmax_b))
        out_v[pl.ds(dv * _L, _L)] = jnp.where(sel, 1.0, 0.0)
    pltpu.sync_copy(out_v, out_hbm.at[pl.ds(base, _DPW)])


def _poly_mask_input_kernel(coef_ref, mask_ref, x_ref, prev_ref, o_ref):
    del prev_ref
    x = x_ref[...]
    c0 = coef_ref[0]
    c1 = coef_ref[1]
    c2 = coef_ref[2]
    p = x * (c0 + x * (c1 + x * c2))
    m = mask_ref[0:1, :]
    o_ref[...] = jnp.where(m != 0.0, p, x)


def _poly_fused_mask_kernel(coef_ref, imp_row_ref, imp_col_ref, x_ref, o_ref,
                            mask_ref):
    @pl.when(pl.program_id(0) == 0)
    def _compute_mask():
        imp_col = imp_col_ref[:, :]
        e_idx = jax.lax.broadcasted_iota(jnp.int32, (_D, 1), 0)
        chunk = 256
        for c in range(_D // chunk):
            d_vals = imp_row_ref[0:1, c * chunk:(c + 1) * chunk]
            d_idx = jax.lax.broadcasted_iota(
                jnp.int32, (1, chunk), 1) + c * chunk
            gt = jnp.sum((imp_col > d_vals).astype(jnp.float32), axis=0,
                         keepdims=True)
            eq_before = jnp.sum(
                ((imp_col == d_vals) & (e_idx < d_idx)).astype(jnp.float32),
                axis=0, keepdims=True)
            mask_ref[0:1, c * chunk:(c + 1) * chunk] = (
                (gt + eq_before) < float(_KEEP)).astype(jnp.float32)

    x = x_ref[...]
    c0 = coef_ref[0]
    c1 = coef_ref[1]
    c2 = coef_ref[2]
    p = x * (c0 + x * (c1 + x * c2))
    m = mask_ref[0:1, :]
    o_ref[...] = jnp.where(m != 0.0, p, x)


_N_FUSED_BLOCKS = 8


@jax.jit
def kernel(x, coeffs, importance):
    B, T, D = x.shape
    assert D == _D

    xf = x.reshape(B * T, D)
    n_blocks = (B * T) // _ROWS_PER_BLOCK
    n1a = 2
    n1 = _N_FUSED_BLOCKS

    def fused_call(grid_lo, grid_n, prev):
        specs = [
            pl.BlockSpec(memory_space=pltpu.SMEM),
            pl.BlockSpec((1, D), lambda i: (0, 0)),
            pl.BlockSpec((D, 1), lambda i: (0, 0)),
            pl.BlockSpec((_ROWS_PER_BLOCK, D), lambda i: (i + grid_lo, 0)),
        ]
        args = [coeffs, importance.reshape(1, D), importance.reshape(D, 1), xf]
        aliases = {}
        if prev is not None:
            specs.append(pl.BlockSpec(memory_space=pl.ANY))
            args.append(prev)
            aliases = {4: 0}
        body = _poly_fused_mask_kernel if prev is None else (
            lambda c, ir, ic, xr, pr, o, m: _poly_fused_mask_kernel(
                c, ir, ic, xr, o, m))
        return pl.pallas_call(
            body,
            grid=(grid_n,),
            in_specs=specs,
            out_specs=pl.BlockSpec((_ROWS_PER_BLOCK, D),
                                   lambda i: (i + grid_lo, 0)),
            out_shape=jax.ShapeDtypeStruct((B * T, D), jnp.float32),
            scratch_shapes=[pltpu.VMEM((1, D), jnp.float32)],
            input_output_aliases=aliases,
        )(*args)

    out1a = fused_call(0, n1a, None)
    sc_mask = _sc_mask(importance).reshape(1, D)
    out1 = fused_call(n1a, n1 - n1a, out1a)

    out = pl.pallas_call(
        _poly_mask_input_kernel,
        grid=(n_blocks - n1,),
        in_specs=[
            pl.BlockSpec(memory_space=pltpu.SMEM),
            pl.BlockSpec((1, D), lambda i: (0, 0)),
            pl.BlockSpec((_ROWS_PER_BLOCK, D), lambda i: (i + n1, 0)),
            pl.BlockSpec(memory_space=pl.ANY),
        ],
        out_specs=pl.BlockSpec((_ROWS_PER_BLOCK, D), lambda i: (i + n1, 0)),
        out_shape=jax.ShapeDtypeStruct((B * T, D), jnp.float32),
        input_output_aliases={3: 0},
    )(coeffs, sc_mask, xf, out1)

    return out.reshape(B, T, D)


# speedup vs baseline: 1.0180x; 1.0180x over previous
import functools

import jax
import jax.numpy as jnp
from jax import lax
from jax.experimental import pallas as pl
from jax.experimental.pallas import tpu as pltpu
from jax.experimental.pallas import tpu_sc as plsc

_D = 2048
_KEEP = max(1, int(_D * 0.5))
_ROWS_PER_BLOCK = 1024

_NC = 1
_NS = 16
_L = 16
_NW = _NC * _NS
_DPW = _D // _NW
_NDV = _DPW // _L


_NCHUNK = _D // _L


@functools.partial(
    pl.kernel,
    out_type=jax.ShapeDtypeStruct((_D,), jnp.float32),
    mesh=plsc.VectorSubcoreMesh(core_axis_name="c", subcore_axis_name="s",
                                num_cores=_NC),
    scratch_types=[
        pltpu.VMEM((_D,), jnp.float32),
        pltpu.VMEM((_D,), jnp.uint32),
        pltpu.VMEM((_DPW,), jnp.float32),
    ],
)
def _sc_mask(imp_hbm, out_hbm, imp_v, key_v, out_v):
    wid = lax.axis_index("s") * _NC + lax.axis_index("c")
    base = wid * _DPW
    pltpu.sync_copy(imp_hbm, imp_v)

    lane = lax.iota(jnp.int32, _L)

    def vsum(v):
        parts = [v[l] for l in range(_L)]
        while len(parts) > 1:
            parts = [parts[i] + parts[i + 1]
                     for i in range(0, len(parts) - 1, 2)] + (
                         [parts[-1]] if len(parts) % 2 else [])
        return parts[0]

    def to_key(c, _):
        v = imp_v[pl.ds(c * _L, _L)] + 0.0
        b = lax.bitcast_convert_type(v, jnp.int32)
        ks = b ^ (jnp.uint32(0x7FFFFFFF).astype(jnp.int32) & (b >> 31))
        key_v[pl.ds(c * _L, _L)] = lax.bitcast_convert_type(
            ks, jnp.uint32) ^ jnp.uint32(0x80000000)
        return 0

    lax.fori_loop(0, _NCHUNK, to_key, 0)

    _UNROLL = 8

    def count_chunks(hit_fn):
        def cbody(i, accs):
            return tuple(
                accs[u] + jnp.where(hit_fn(i * _UNROLL + u), 1, 0).astype(
                    jnp.int32) for u in range(_UNROLL))

        accs = lax.fori_loop(0, _NCHUNK // _UNROLL, cbody,
                             (jnp.zeros((_L,), jnp.int32),) * _UNROLL)
        accs = list(accs)
        while len(accs) > 1:
            accs = [accs[i] + accs[i + 1] for i in range(0, len(accs), 2)]
        return vsum(accs[0])

    def count_ge(thr):
        thr_b = jnp.full((_L,), thr, jnp.uint32)
        return count_chunks(lambda c: key_v[pl.ds(c * _L, _L)] >= thr_b)

    def round_(r, prefix):
        bit = 31 - r
        cand = prefix | (jnp.uint32(1) << bit.astype(jnp.uint32))
        cnt = count_ge(cand)
        return jnp.where(cnt >= _KEEP, cand, prefix)

    thr = lax.fori_loop(0, 32, round_, jnp.uint32(0))

    thr_b = jnp.full((_L,), thr, jnp.uint32)

    cnt_gt = count_chunks(lambda c: key_v[pl.ds(c * _L, _L)] > thr_b)
    quota = _KEEP - cnt_gt

    def count_eq_below(c):
        c_b = jnp.full((_L,), c, jnp.int32)

        def hit(ch):
            kc = key_v[pl.ds(ch * _L, _L)]
            eidx = lane + ch * _L
            return (kc == thr_b) & (eidx < c_b)

        return count_chunks(hit)

    def idx_round(r, cmax):
        bit = 10 - r
        cand = cmax | (jnp.int32(1) << bit)
        cnt = count_eq_below(cand)
        return jnp.where(cnt < quota, cand, cmax)

    cmax = lax.fori_loop(0, 11, idx_round, jnp.int32(0))

    cmax_b = jnp.full((_L,), cmax, jnp.int32)
    for dv in range(_NDV):
        kdv = key_v[pl.ds(base + dv * _L, _L)]
        didx = lane + (base + dv * _L)
        sel = (kdv > thr_b) | ((kdv == thr_b) & (didx <= cmax_b))
        out_v[pl.ds(dv * _L, _L)] = jnp.where(sel, 1.0, 0.0)
    pltpu.sync_copy(out_v, out_hbm.at[pl.ds(base, _DPW)])


def _poly_mask_input_kernel(coef_ref, mask_ref, x_ref, o_ref):
    x = x_ref[...]
    c0 = coef_ref[0]
    c1 = coef_ref[1]
    c2 = coef_ref[2]
    p = x * (c0 + x * (c1 + x * c2))
    m = mask_ref[0:1, :]
    o_ref[...] = jnp.where(m != 0.0, p, x)


@jax.jit
def kernel(x, coeffs, importance):
    B, T, D = x.shape
    assert D == _D

    sc_mask = _sc_mask(importance).reshape(1, D)

    xf = x.reshape(B * T, D)
    n_blocks = (B * T) // _ROWS_PER_BLOCK
    out = pl.pallas_call(
        _poly_mask_input_kernel,
        grid=(n_blocks,),
        in_specs=[
            pl.BlockSpec(memory_space=pltpu.SMEM),
            pl.BlockSpec((1, D), lambda i: (0, 0)),
            pl.BlockSpec((_ROWS_PER_BLOCK, D), lambda i: (i, 0)),
        ],
        out_specs=pl.BlockSpec((_ROWS_PER_BLOCK, D), lambda i: (i, 0)),
        out_shape=jax.ShapeDtypeStruct((B * T, D), jnp.float32),
    )(coeffs, sc_mask, xf)
    return out.reshape(B, T, D)


# speedup vs baseline: 1.0529x; 1.0343x over previous
import functools

import jax
import jax.numpy as jnp
from jax import lax
from jax.experimental import pallas as pl
from jax.experimental.pallas import tpu as pltpu
from jax.experimental.pallas import tpu_sc as plsc

_D = 2048
_KEEP = max(1, int(_D * 0.5))
_ROWS_PER_BLOCK = 1024

_NC = 1
_NS = 16
_L = 16
_NW = _NC * _NS
_DPW = _D // _NW
_NDV = _DPW // _L
_NCHUNK = _D // _L


@functools.partial(
    pl.kernel,
    out_type=jax.ShapeDtypeStruct((_D,), jnp.float32),
    mesh=plsc.VectorSubcoreMesh(core_axis_name="c", subcore_axis_name="s",
                                num_cores=_NC),
    scratch_types=[
        pltpu.VMEM((_D,), jnp.float32),
        pltpu.VMEM((_D,), jnp.uint32),
        pltpu.VMEM((_DPW,), jnp.float32),
    ],
)
def _sc_mask(imp_hbm, out_hbm, imp_v, key_v, out_v):
    wid = lax.axis_index("s") * _NC + lax.axis_index("c")
    base = wid * _DPW
    pltpu.sync_copy(imp_hbm, imp_v)

    lane = lax.iota(jnp.int32, _L)

    def vsum(v):
        parts = [v[l] for l in range(_L)]
        while len(parts) > 1:
            parts = [parts[i] + parts[i + 1]
                     for i in range(0, len(parts) - 1, 2)] + (
                         [parts[-1]] if len(parts) % 2 else [])
        return parts[0]

    def to_key(c, _):
        v = imp_v[pl.ds(c * _L, _L)] + 0.0
        b = lax.bitcast_convert_type(v, jnp.int32)
        ks = b ^ (jnp.uint32(0x7FFFFFFF).astype(jnp.int32) & (b >> 31))
        key_v[pl.ds(c * _L, _L)] = lax.bitcast_convert_type(
            ks, jnp.uint32) ^ jnp.uint32(0x80000000)
        return 0

    lax.fori_loop(0, _NCHUNK, to_key, 0)

    _UNROLL = 8

    def count3(hit3_fn):
        def cbody(i, accs):
            a = list(accs)
            for u in range(_UNROLL):
                h1, h2, h3 = hit3_fn(i * _UNROLL + u)
                j = 3 * (u % 2)
                a[j] = a[j] + jnp.where(h1, 1, 0).astype(jnp.int32)
                a[j + 1] = a[j + 1] + jnp.where(h2, 1, 0).astype(jnp.int32)
                a[j + 2] = a[j + 2] + jnp.where(h3, 1, 0).astype(jnp.int32)
            return tuple(a)

        z = jnp.zeros((_L,), jnp.int32)
        accs = lax.fori_loop(0, _NCHUNK // _UNROLL, cbody, (z,) * 6)
        return (vsum(accs[0] + accs[3]), vsum(accs[1] + accs[4]),
                vsum(accs[2] + accs[5]))

    def round_(r, prefix):
        shift = (30 - 2 * r).astype(jnp.uint32)
        c1 = prefix | (jnp.uint32(1) << shift)
        c2 = prefix | (jnp.uint32(2) << shift)
        c3 = prefix | (jnp.uint32(3) << shift)

        def hit3(c):
            kc = key_v[pl.ds(c * _L, _L)]
            return (kc >= jnp.full((_L,), c1, jnp.uint32),
                    kc >= jnp.full((_L,), c2, jnp.uint32),
                    kc >= jnp.full((_L,), c3, jnp.uint32))

        n1, n2, n3 = count3(hit3)
        prefix = jnp.where(n1 >= _KEEP, c1, prefix)
        prefix = jnp.where(n2 >= _KEEP, c2, prefix)
        prefix = jnp.where(n3 >= _KEEP, c3, prefix)
        return prefix

    thr = lax.fori_loop(0, 16, round_, jnp.uint32(0))

    thr_b = jnp.full((_L,), thr, jnp.uint32)

    def hit_gt(c):
        kc = key_v[pl.ds(c * _L, _L)]
        g = kc > thr_b
        return (g, g, g)

    cnt_gt, _, _ = count3(hit_gt)
    quota = _KEEP - cnt_gt

    def idx_round(r, cmax):
        shift = 10 - 2 * r
        c1 = cmax | (jnp.int32(1) << shift)
        c2 = cmax | (jnp.int32(2) << shift)
        c3 = cmax | (jnp.int32(3) << shift)

        def hit3(ch):
            kc = key_v[pl.ds(ch * _L, _L)]
            eidx = lane + ch * _L
            eq = kc == thr_b
            return (eq & (eidx < jnp.full((_L,), c1, jnp.int32)),
                    eq & (eidx < jnp.full((_L,), c2, jnp.int32)),
                    eq & (eidx < jnp.full((_L,), c3, jnp.int32)))

        n1, n2, n3 = count3(hit3)
        cmax = jnp.where(n1 < quota, c1, cmax)
        cmax = jnp.where(n2 < quota, c2, cmax)
        cmax = jnp.where(n3 < quota, c3, cmax)
        return cmax

    cmax = lax.fori_loop(0, 6, idx_round, jnp.int32(0))

    cmax_b = jnp.full((_L,), cmax, jnp.int32)
    for dv in range(_NDV):
        kdv = key_v[pl.ds(base + dv * _L, _L)]
        didx = lane + (base + dv * _L)
        sel = (kdv > thr_b) | ((kdv == thr_b) & (didx <= cmax_b))
        out_v[pl.ds(dv * _L, _L)] = jnp.where(sel, 1.0, 0.0)
    pltpu.sync_copy(out_v, out_hbm.at[pl.ds(base, _DPW)])


def _poly_mask_input_kernel(coef_ref, mask_ref, x_ref, prev_ref, o_ref):
    del prev_ref
    x = x_ref[...]
    c0 = coef_ref[0]
    c1 = coef_ref[1]
    c2 = coef_ref[2]
    p = x * (c0 + x * (c1 + x * c2))
    m = mask_ref[0:1, :]
    o_ref[...] = jnp.where(m != 0.0, p, x)


def _poly_fused_mask_kernel(coef_ref, imp_row_ref, imp_col_ref, x_ref, o_ref,
                            mask_ref):
    @pl.when(pl.program_id(0) == 0)
    def _compute_mask():
        imp_col = imp_col_ref[:, :]
        e_idx = jax.lax.broadcasted_iota(jnp.int32, (_D, 1), 0)
        chunk = 256
        for c in range(_D // chunk):
            d_vals = imp_row_ref[0:1, c * chunk:(c + 1) * chunk]
            d_idx = jax.lax.broadcasted_iota(
                jnp.int32, (1, chunk), 1) + c * chunk
            gt = jnp.sum((imp_col > d_vals).astype(jnp.float32), axis=0,
                         keepdims=True)
            eq_before = jnp.sum(
                ((imp_col == d_vals) & (e_idx < d_idx)).astype(jnp.float32),
                axis=0, keepdims=True)
            mask_ref[0:1, c * chunk:(c + 1) * chunk] = (
                (gt + eq_before) < float(_KEEP)).astype(jnp.float32)

    x = x_ref[...]
    c0 = coef_ref[0]
    c1 = coef_ref[1]
    c2 = coef_ref[2]
    p = x * (c0 + x * (c1 + x * c2))
    m = mask_ref[0:1, :]
    o_ref[...] = jnp.where(m != 0.0, p, x)


_N_FUSED_BLOCKS = 8


@jax.jit
def kernel(x, coeffs, importance):
    B, T, D = x.shape
    assert D == _D

    sc_mask = _sc_mask(importance).reshape(1, D)

    xf = x.reshape(B * T, D)
    n_blocks = (B * T) // _ROWS_PER_BLOCK
    n1 = _N_FUSED_BLOCKS

    out1 = pl.pallas_call(
        _poly_fused_mask_kernel,
        grid=(n1,),
        in_specs=[
            pl.BlockSpec(memory_space=pltpu.SMEM),
            pl.BlockSpec((1, D), lambda i: (0, 0)),
            pl.BlockSpec((D, 1), lambda i: (0, 0)),
            pl.BlockSpec((_ROWS_PER_BLOCK, D), lambda i: (i, 0)),
        ],
        out_specs=pl.BlockSpec((_ROWS_PER_BLOCK, D), lambda i: (i, 0)),
        out_shape=jax.ShapeDtypeStruct((B * T, D), jnp.float32),
        scratch_shapes=[pltpu.VMEM((1, D), jnp.float32)],
    )(coeffs, importance.reshape(1, D), importance.reshape(D, 1), xf)

    out = pl.pallas_call(
        _poly_mask_input_kernel,
        grid=(n_blocks - n1,),
        in_specs=[
            pl.BlockSpec(memory_space=pltpu.SMEM),
            pl.BlockSpec((1, D), lambda i: (0, 0)),
            pl.BlockSpec((_ROWS_PER_BLOCK, D), lambda i: (i + n1, 0)),
            pl.BlockSpec(memory_space=pl.ANY),
        ],
        out_specs=pl.BlockSpec((_ROWS_PER_BLOCK, D), lambda i: (i + n1, 0)),
        out_shape=jax.ShapeDtypeStruct((B * T, D), jnp.float32),
        input_output_aliases={3: 0},
    )(coeffs, sc_mask, xf, out1)

    return out.reshape(B, T, D)
